# baseline (speedup 1.0000x reference)
import functools

import jax
import jax.numpy as jnp
from jax import lax
from jax.experimental import pallas as pl
from jax.experimental.pallas import tpu as pltpu
from jax.experimental.pallas import tpu_sc as plsc

NUM_EMB = 1000000
DIM = 32
N_B = 4096
N_T = 200
NC, NS = 2, 16
NW = NC * NS
B_BLK = N_B // NW
T_CHUNK = 10
NSTEPS = N_T // T_CHUNK

_mesh = plsc.VectorSubcoreMesh(
    core_axis_name="c", subcore_axis_name="s", num_cores=NC, num_subcores=NS
)


@functools.partial(
    pl.kernel,
    out_type=jax.ShapeDtypeStruct((N_B, N_T, DIM), jnp.float32),
    mesh=_mesh,
    compiler_params=pltpu.CompilerParams(use_tc_tiling_on_sc=False),
    scratch_types=[
        pltpu.VMEM((N_T, B_BLK), jnp.int32),
        pltpu.VMEM((T_CHUNK, B_BLK, DIM), jnp.float32),
        pltpu.VMEM((T_CHUNK, B_BLK, DIM), jnp.float32),
        pltpu.SemaphoreType.DMA,
        pltpu.SemaphoreType.DMA,
        pltpu.SemaphoreType.DMA,
        pltpu.SemaphoreType.DMA,
    ],
)
def _emb_lookup(idx_hbm, table_hbm, out_hbm, idx_v, buf0, buf1,
                g0, g1, w0, w1):
    wid = lax.axis_index("s") * NC + lax.axis_index("c")
    b0 = wid * B_BLK
    pltpu.sync_copy(idx_hbm.at[:, pl.ds(b0, B_BLK)], idx_v)

    def fire(c, buf, sem):
        for tt in range(T_CHUNK):
            pltpu.async_copy(
                table_hbm.at[idx_v.at[c * T_CHUNK + tt]], buf.at[tt], sem
            )

    def drain(c, buf, sem):
        for tt in range(T_CHUNK):
            pltpu.make_async_copy(
                table_hbm.at[idx_v.at[c * T_CHUNK + tt]], buf.at[tt], sem
            ).wait()

    def writeback(c, buf, sem):
        @pl.loop(0, B_BLK)
        def _wb(bb):
            pltpu.async_copy(
                buf.at[:, bb, :],
                out_hbm.at[b0 + bb, pl.ds(c * T_CHUNK, T_CHUNK), :],
                sem,
            )

    def drain_wb(c, buf, sem):
        @pl.loop(0, B_BLK)
        def _dw(bb):
            pltpu.make_async_copy(
                buf.at[:, bb, :],
                out_hbm.at[b0 + bb, pl.ds(c * T_CHUNK, T_CHUNK), :],
                sem,
            ).wait()

    fire(0, buf0, g0)
    fire(1, buf1, g1)

    @pl.loop(0, NSTEPS // 2)
    def _pair(j):
        c0 = 2 * j
        drain(c0, buf0, g0)
        writeback(c0, buf0, w0)
        drain_wb(c0, buf0, w0)

        @pl.when(c0 + 2 < NSTEPS)
        def _():
            fire(c0 + 2, buf0, g0)

        drain(c0 + 1, buf1, g1)
        writeback(c0 + 1, buf1, w1)
        drain_wb(c0 + 1, buf1, w1)

        @pl.when(c0 + 3 < NSTEPS)
        def _():
            fire(c0 + 3, buf1, g1)


def kernel(tokens, weight):
    tok_t = tokens.T.astype(jnp.int32)
    return _emb_lookup(tok_t, weight)


# speedup vs baseline: 1.3160x; 1.3160x over previous
import functools

import jax
import jax.numpy as jnp
from jax import lax
from jax.experimental import pallas as pl
from jax.experimental.pallas import tpu as pltpu
from jax.experimental.pallas import tpu_sc as plsc

NUM_EMB = 1000000
DIM = 32
N_B = 4096
N_T = 200
NC, NS = 2, 16
NW = NC * NS
B_BLK = N_B // NW
T_CHUNK = 10
NSTEPS = N_T // T_CHUNK

E4 = 2048
GRP = 4 * E4
PRE_GRID = -(-NUM_EMB // GRP)
W_ROWS = PRE_GRID * E4
V_PAD = W_ROWS * 4


def _pre_body(wt_ref, out_ref):
    x = wt_ref[...]
    out_ref[...] = jnp.concatenate(
        [jnp.transpose(x[:, q * E4:(q + 1) * E4]) for q in range(4)], axis=1
    )


_pre_transpose = pl.pallas_call(
    _pre_body,
    grid=(PRE_GRID,),
    in_specs=[pl.BlockSpec((DIM, GRP), lambda i: (0, i))],
    out_specs=pl.BlockSpec((E4, 128), lambda i: (i, 0)),
    out_shape=jax.ShapeDtypeStruct((W_ROWS, 128), jnp.float32),
)

_mesh = plsc.VectorSubcoreMesh(
    core_axis_name="c", subcore_axis_name="s", num_cores=NC, num_subcores=NS
)


@functools.partial(
    pl.kernel,
    out_type=jax.ShapeDtypeStruct((N_B, N_T, DIM), jnp.float32),
    mesh=_mesh,
    compiler_params=pltpu.CompilerParams(use_tc_tiling_on_sc=False),
    scratch_types=[
        pltpu.VMEM((N_T, B_BLK), jnp.int32),
        pltpu.VMEM((T_CHUNK, B_BLK, DIM), jnp.float32),
        pltpu.VMEM((T_CHUNK, B_BLK, DIM), jnp.float32),
        pltpu.SemaphoreType.DMA,
        pltpu.SemaphoreType.DMA,
        pltpu.SemaphoreType.DMA,
        pltpu.SemaphoreType.DMA,
    ],
)
def _emb_lookup(idx_hbm, table_hbm, out_hbm, idx_v, buf0, buf1,
                g0, g1, w0, w1):
    wid = lax.axis_index("s") * NC + lax.axis_index("c")
    b0 = wid * B_BLK
    pltpu.sync_copy(idx_hbm.at[:, pl.ds(b0, B_BLK)], idx_v)

    @pl.loop(0, N_T)
    def _fix(r):
        for c in range(B_BLK // 16):
            e = idx_v[r, pl.ds(c * 16, 16)]
            m = (e & ~(GRP - 1)) | ((e & (E4 - 1)) << 2) | ((e >> 11) & 3)
            idx_v[r, pl.ds(c * 16, 16)] = m

    def fire(c, buf, sem):
        for tt in range(T_CHUNK):
            pltpu.async_copy(
                table_hbm.at[idx_v.at[c * T_CHUNK + tt]], buf.at[tt], sem
            )

    def drain(c, buf, sem):
        for tt in range(T_CHUNK):
            pltpu.make_async_copy(
                table_hbm.at[idx_v.at[c * T_CHUNK + tt]], buf.at[tt], sem
            ).wait()

    def writeback(c, buf, sem):
        @pl.loop(0, B_BLK)
        def _wb(bb):
            pltpu.async_copy(
                buf.at[:, bb, :],
                out_hbm.at[b0 + bb, pl.ds(c * T_CHUNK, T_CHUNK), :],
                sem,
            )

    def drain_wb(c, buf, sem):
        @pl.loop(0, B_BLK)
        def _dw(bb):
            pltpu.make_async_copy(
                buf.at[:, bb, :],
                out_hbm.at[b0 + bb, pl.ds(c * T_CHUNK, T_CHUNK), :],
                sem,
            ).wait()

    fire(0, buf0, g0)
    fire(1, buf1, g1)

    @pl.loop(0, NSTEPS // 2)
    def _pair(j):
        c0 = 2 * j
        drain(c0, buf0, g0)
        writeback(c0, buf0, w0)
        drain_wb(c0, buf0, w0)

        @pl.when(c0 + 2 < NSTEPS)
        def _():
            fire(c0 + 2, buf0, g0)

        drain(c0 + 1, buf1, g1)
        writeback(c0 + 1, buf1, w1)
        drain_wb(c0 + 1, buf1, w1)

        @pl.when(c0 + 3 < NSTEPS)
        def _():
            fire(c0 + 3, buf1, g1)


def kernel(tokens, weight):
    tok_t = tokens.T.astype(jnp.int32)
    w128 = _pre_transpose(weight.T)
    return _emb_lookup(tok_t, w128.reshape(V_PAD, DIM))


# speedup vs baseline: 1.9777x; 1.5029x over previous
import functools

import jax
import jax.numpy as jnp
from jax import lax
from jax.experimental import pallas as pl
from jax.experimental.pallas import tpu as pltpu
from jax.experimental.pallas import tpu_sc as plsc

NUM_EMB = 1000000
DIM = 32
N_B = 4096
N_T = 200
NC, NS = 2, 16
NW = NC * NS
B_BLK = N_B // NW
T_CHUNK = 10
NSTEPS = N_T // T_CHUNK

E4 = 2048
GRP = 4 * E4
PRE_GRID = -(-NUM_EMB // GRP)
W_ROWS = PRE_GRID * E4
V_PAD = W_ROWS * 4


def _pre_body(wt_ref, out_ref):
    x = wt_ref[...]
    out_ref[...] = jnp.concatenate(
        [jnp.transpose(x[:, q * E4:(q + 1) * E4]) for q in range(4)], axis=1
    )


_pre_transpose = pl.pallas_call(
    _pre_body,
    grid=(PRE_GRID,),
    in_specs=[pl.BlockSpec((DIM, GRP), lambda i: (0, i))],
    out_specs=pl.BlockSpec((E4, 128), lambda i: (i, 0)),
    out_shape=jax.ShapeDtypeStruct((W_ROWS, 128), jnp.float32),
)

_mesh = plsc.VectorSubcoreMesh(
    core_axis_name="c", subcore_axis_name="s", num_cores=NC, num_subcores=NS
)


@functools.partial(
    pl.kernel,
    out_type=jax.ShapeDtypeStruct((N_T, 8, B_BLK, 4, DIM), jnp.float32),
    mesh=_mesh,
    compiler_params=pltpu.CompilerParams(use_tc_tiling_on_sc=False),
    scratch_types=[
        pltpu.VMEM((N_T, B_BLK), jnp.int32),
        pltpu.VMEM((T_CHUNK, B_BLK, DIM), jnp.float32),
        pltpu.VMEM((T_CHUNK, B_BLK, DIM), jnp.float32),
        pltpu.SemaphoreType.DMA,
        pltpu.SemaphoreType.DMA,
        pltpu.SemaphoreType.DMA,
        pltpu.SemaphoreType.DMA,
    ],
)
def _emb_lookup(idx_hbm, table_hbm, out_hbm, idx_v, buf0, buf1,
                g0, g1, w0, w1):
    wid = lax.axis_index("s") * NC + lax.axis_index("c")
    b0 = wid * B_BLK
    jq = wid // 4
    uq = lax.rem(wid, 4)
    pltpu.sync_copy(idx_hbm.at[:, pl.ds(b0, B_BLK)], idx_v)

    @pl.loop(0, N_T)
    def _fix(r):
        for c in range(B_BLK // 16):
            e = idx_v[r, pl.ds(c * 16, 16)]
            m = (e & ~(GRP - 1)) | ((e & (E4 - 1)) << 2) | ((e >> 11) & 3)
            idx_v[r, pl.ds(c * 16, 16)] = m

    def fire(c, buf, sem):
        for tt in range(T_CHUNK):
            pltpu.async_copy(
                table_hbm.at[idx_v.at[c * T_CHUNK + tt]], buf.at[tt], sem
            )

    def drain(c, buf, sem):
        for tt in range(T_CHUNK):
            pltpu.make_async_copy(
                table_hbm.at[idx_v.at[c * T_CHUNK + tt]], buf.at[tt], sem
            ).wait()

    def wb_copy(c, buf, sem):
        return pltpu.make_async_copy(
            buf,
            out_hbm.at[pl.ds(c * T_CHUNK, T_CHUNK), jq, :, uq, :],
            sem,
        )

    def writeback(c, buf, sem):
        wb_copy(c, buf, sem).start()

    def drain_wb(c, buf, sem):
        wb_copy(c, buf, sem).wait()

    fire(0, buf0, g0)
    fire(1, buf1, g1)

    @pl.loop(0, NSTEPS // 2)
    def _pair(j):
        c0 = 2 * j
        drain(c0, buf0, g0)
        writeback(c0, buf0, w0)
        drain_wb(c0, buf0, w0)

        @pl.when(c0 + 2 < NSTEPS)
        def _():
            fire(c0 + 2, buf0, g0)

        drain(c0 + 1, buf1, g1)
        writeback(c0 + 1, buf1, w1)
        drain_wb(c0 + 1, buf1, w1)

        @pl.when(c0 + 3 < NSTEPS)
        def _():
            fire(c0 + 3, buf1, g1)


def _post_body(x_ref, out_ref):
    z = jnp.transpose(x_ref[...])
    y = jnp.concatenate(
        [z[32 * u:32 * (u + 1), 128 * j:128 * (j + 1)]
         for j in range(8) for u in range(4)],
        axis=1,
    )
    out_ref[...] = y[None]


_post_transpose = pl.pallas_call(
    _post_body,
    grid=(N_T,),
    in_specs=[pl.BlockSpec((1024, 128), lambda t: (t, 0))],
    out_specs=pl.BlockSpec((1, DIM, N_B), lambda t: (t, 0, 0)),
    out_shape=jax.ShapeDtypeStruct((N_T, DIM, N_B), jnp.float32),
)


def kernel(tokens, weight):
    tok_t = tokens.T.astype(jnp.int32)
    w128 = _pre_transpose(weight.T)
    g4 = _emb_lookup(tok_t, w128.reshape(V_PAD, DIM))
    out_perm = _post_transpose(g4.reshape(N_T * 8 * B_BLK, 128))
    return jnp.transpose(out_perm, (2, 0, 1))


# speedup vs baseline: 2.5169x; 1.2726x over previous
import functools

import jax
import jax.numpy as jnp
from jax import lax
from jax.experimental import pallas as pl
from jax.experimental.pallas import tpu as pltpu
from jax.experimental.pallas import tpu_sc as plsc

NUM_EMB = 1000000
DIM = 32
N_B = 4096
N_T = 200
NC, NS = 2, 16
NW = NC * NS
B_BLK = N_B // NW
T_CHUNK = 10
NSTEPS = N_T // T_CHUNK

E4 = 2048
GRP = 4 * E4
PRE_GRID = -(-NUM_EMB // GRP)
W_ROWS = PRE_GRID * E4
V_PAD = W_ROWS * 4


def _pre_body(wt_ref, out_ref):
    x = wt_ref[...]
    xx = jnp.concatenate(
        [x[:, q * E4:(q + 1) * E4] for q in range(4)], axis=0
    )
    out_ref[...] = jnp.transpose(xx)


_pre_transpose = pl.pallas_call(
    _pre_body,
    grid=(PRE_GRID,),
    in_specs=[pl.BlockSpec((DIM, GRP), lambda i: (0, i))],
    out_specs=pl.BlockSpec((E4, 128), lambda i: (i, 0)),
    out_shape=jax.ShapeDtypeStruct((W_ROWS, 128), jnp.float32),
)

_mesh = plsc.VectorSubcoreMesh(
    core_axis_name="c", subcore_axis_name="s", num_cores=NC, num_subcores=NS
)


@functools.partial(
    pl.kernel,
    out_type=jax.ShapeDtypeStruct((N_T, 8, B_BLK, 4, DIM), jnp.float32),
    mesh=_mesh,
    compiler_params=pltpu.CompilerParams(use_tc_tiling_on_sc=False),
    scratch_types=[
        pltpu.VMEM((N_T, B_BLK), jnp.int32),
        pltpu.VMEM((T_CHUNK, B_BLK, DIM), jnp.float32),
        pltpu.VMEM((T_CHUNK, B_BLK, DIM), jnp.float32),
        pltpu.SemaphoreType.DMA,
        pltpu.SemaphoreType.DMA,
        pltpu.SemaphoreType.DMA,
        pltpu.SemaphoreType.DMA,
    ],
)
def _emb_lookup(idx_hbm, table_hbm, out_hbm, idx_v, buf0, buf1,
                g0, g1, w0, w1):
    wid = lax.axis_index("s") * NC + lax.axis_index("c")
    b0 = wid * B_BLK
    jq = wid // 4
    uq = lax.rem(wid, 4)
    pltpu.sync_copy(idx_hbm.at[:, pl.ds(b0, B_BLK)], idx_v)

    @pl.loop(0, N_T)
    def _fix(r):
        for c in range(B_BLK // 16):
            e = idx_v[r, pl.ds(c * 16, 16)]
            m = (e & ~(GRP - 1)) | ((e & (E4 - 1)) << 2) | ((e >> 11) & 3)
            idx_v[r, pl.ds(c * 16, 16)] = m

    def fire(c, buf, sem):
        for tt in range(T_CHUNK):
            pltpu.async_copy(
                table_hbm.at[idx_v.at[c * T_CHUNK + tt]], buf.at[tt], sem
            )

    def drain(c, buf, sem):
        for tt in range(T_CHUNK):
            pltpu.make_async_copy(
                table_hbm.at[idx_v.at[c * T_CHUNK + tt]], buf.at[tt], sem
            ).wait()

    def wb_copy(c, buf, sem):
        return pltpu.make_async_copy(
            buf,
            out_hbm.at[pl.ds(c * T_CHUNK, T_CHUNK), jq, :, uq, :],
            sem,
        )

    def writeback(c, buf, sem):
        wb_copy(c, buf, sem).start()

    def drain_wb(c, buf, sem):
        wb_copy(c, buf, sem).wait()

    fire(0, buf0, g0)
    fire(1, buf1, g1)

    @pl.loop(0, NSTEPS // 2)
    def _pair(j):
        c0 = 2 * j
        drain(c0, buf0, g0)
        writeback(c0, buf0, w0)
        drain_wb(c0, buf0, w0)

        @pl.when(c0 + 2 < NSTEPS)
        def _():
            fire(c0 + 2, buf0, g0)

        drain(c0 + 1, buf1, g1)
        writeback(c0 + 1, buf1, w1)
        drain_wb(c0 + 1, buf1, w1)

        @pl.when(c0 + 3 < NSTEPS)
        def _():
            fire(c0 + 3, buf1, g1)


def _post_body(x_ref, out_ref):
    z = jnp.transpose(x_ref[...])
    y = jnp.concatenate(
        [z[32 * u:32 * (u + 1), 128 * j:128 * (j + 1)]
         for j in range(8) for u in range(4)],
        axis=1,
    )
    out_ref[...] = y[None]


_post_transpose = pl.pallas_call(
    _post_body,
    grid=(N_T,),
    in_specs=[pl.BlockSpec((1024, 128), lambda t: (t, 0))],
    out_specs=pl.BlockSpec((1, DIM, N_B), lambda t: (t, 0, 0)),
    out_shape=jax.ShapeDtypeStruct((N_T, DIM, N_B), jnp.float32),
)


def kernel(tokens, weight):
    tok_t = tokens.T.astype(jnp.int32)
    w128 = _pre_transpose(weight.T)
    g4 = _emb_lookup(tok_t, w128.reshape(V_PAD, DIM))
    out_perm = _post_transpose(g4.reshape(N_T * 8 * B_BLK, 128))
    return jnp.transpose(out_perm, (2, 0, 1))


# speedup vs baseline: 3.2178x; 1.2785x over previous
import functools

import jax
import jax.numpy as jnp
from jax import lax
from jax.experimental import pallas as pl
from jax.experimental.pallas import tpu as pltpu
from jax.experimental.pallas import tpu_sc as plsc

NUM_EMB = 1000000
DIM = 32
N_B = 4096
N_T = 200
NC, NS = 2, 16
NW = NC * NS
B_BLK = N_B // NW
T_CHUNK = 10
NSTEPS = N_T // T_CHUNK

E4 = 2048
GRP = 4 * E4
PRE_GRID = 2 * (-(-NUM_EMB // (2 * GRP)))
W_ROWS = PRE_GRID * E4
V_PAD = W_ROWS * 4


def _pre_body(wt_ref, out_ref):
    x = wt_ref[...]
    zs = []
    for h in range(2):
        xh = x[:, h * GRP:(h + 1) * GRP]
        xx = jnp.concatenate(
            [xh[:, q * E4:(q + 1) * E4] for q in range(4)], axis=0
        )
        zs.append(jnp.transpose(xx))
    out_ref[...] = jnp.concatenate(zs, axis=0)


_pre_transpose = pl.pallas_call(
    _pre_body,
    grid=(PRE_GRID // 2,),
    in_specs=[pl.BlockSpec((DIM, 2 * GRP), lambda i: (0, i))],
    out_specs=pl.BlockSpec((2 * E4, 128), lambda i: (i, 0)),
    out_shape=jax.ShapeDtypeStruct((W_ROWS, 128), jnp.float32),
)

_mesh = plsc.VectorSubcoreMesh(
    core_axis_name="c", subcore_axis_name="s", num_cores=NC, num_subcores=NS
)


@functools.partial(
    pl.kernel,
    out_type=jax.ShapeDtypeStruct((N_T, 8, B_BLK, 4, DIM), jnp.float32),
    mesh=_mesh,
    compiler_params=pltpu.CompilerParams(use_tc_tiling_on_sc=False),
    scratch_types=[
        pltpu.VMEM((N_T, B_BLK), jnp.int32),
        pltpu.VMEM((T_CHUNK, B_BLK, DIM), jnp.float32),
        pltpu.VMEM((T_CHUNK, B_BLK, DIM), jnp.float32),
        pltpu.SemaphoreType.DMA,
        pltpu.SemaphoreType.DMA,
        pltpu.SemaphoreType.DMA,
        pltpu.SemaphoreType.DMA,
    ],
)
def _emb_lookup(idx_hbm, table_hbm, out_hbm, idx_v, buf0, buf1,
                g0, g1, w0, w1):
    wid = lax.axis_index("s") * NC + lax.axis_index("c")
    b0 = wid * B_BLK
    jq = wid // 4
    uq = lax.rem(wid, 4)
    pltpu.sync_copy(idx_hbm.at[:, pl.ds(b0, B_BLK)], idx_v)

    @pl.loop(0, N_T)
    def _fix(r):
        for c in range(B_BLK // 16):
            e = idx_v[r, pl.ds(c * 16, 16)]
            m = (e & ~(GRP - 1)) | ((e & (E4 - 1)) << 2) | ((e >> 11) & 3)
            idx_v[r, pl.ds(c * 16, 16)] = m

    def fire(c, buf, sem):
        for tt in range(T_CHUNK):
            pltpu.async_copy(
                table_hbm.at[idx_v.at[c * T_CHUNK + tt]], buf.at[tt], sem
            )

    def drain(c, buf, sem):
        for tt in range(T_CHUNK):
            pltpu.make_async_copy(
                table_hbm.at[idx_v.at[c * T_CHUNK + tt]], buf.at[tt], sem
            ).wait()

    def wb_copy(c, buf, sem):
        return pltpu.make_async_copy(
            buf,
            out_hbm.at[pl.ds(c * T_CHUNK, T_CHUNK), jq, :, uq, :],
            sem,
        )

    def writeback(c, buf, sem):
        wb_copy(c, buf, sem).start()

    def drain_wb(c, buf, sem):
        wb_copy(c, buf, sem).wait()

    fire(0, buf0, g0)
    fire(1, buf1, g1)

    @pl.loop(0, NSTEPS // 2)
    def _pair(j):
        c0 = 2 * j
        drain(c0, buf0, g0)
        writeback(c0, buf0, w0)
        drain_wb(c0, buf0, w0)

        @pl.when(c0 + 2 < NSTEPS)
        def _():
            fire(c0 + 2, buf0, g0)

        drain(c0 + 1, buf1, g1)
        writeback(c0 + 1, buf1, w1)
        drain_wb(c0 + 1, buf1, w1)

        @pl.when(c0 + 3 < NSTEPS)
        def _():
            fire(c0 + 3, buf1, g1)


def _post_body(x_ref, out_ref):
    z = jnp.transpose(x_ref[...])
    for tt in range(2):
        y = jnp.concatenate(
            [z[32 * u:32 * (u + 1), 1024 * tt + 128 * j:1024 * tt + 128 * (j + 1)]
             for j in range(8) for u in range(4)],
            axis=1,
        )
        out_ref[tt] = y


_post_transpose = pl.pallas_call(
    _post_body,
    grid=(N_T // 2,),
    in_specs=[pl.BlockSpec((2048, 128), lambda t: (t, 0))],
    out_specs=pl.BlockSpec((2, DIM, N_B), lambda t: (t, 0, 0)),
    out_shape=jax.ShapeDtypeStruct((N_T, DIM, N_B), jnp.float32),
)


def kernel(tokens, weight):
    tok_t = tokens.T.astype(jnp.int32)
    w128 = _pre_transpose(weight.T)
    g4 = _emb_lookup(tok_t, w128.reshape(V_PAD, DIM))
    out_perm = _post_transpose(g4.reshape(N_T * 8 * B_BLK, 128))
    return jnp.transpose(out_perm, (2, 0, 1))


# speedup vs baseline: 3.7624x; 1.1692x over previous
import functools

import jax
import jax.numpy as jnp
from jax import lax
from jax.experimental import pallas as pl
from jax.experimental.pallas import tpu as pltpu
from jax.experimental.pallas import tpu_sc as plsc

NUM_EMB = 1000000
DIM = 32
N_B = 4096
N_T = 200
NC, NS = 2, 16
NW = NC * NS
B_BLK = N_B // NW
T_CHUNK = 10
NSTEPS = N_T // T_CHUNK

E4 = 2048
GRP = 4 * E4
PRE_GRID = 4 * (-(-NUM_EMB // (4 * GRP)))
W_ROWS = PRE_GRID * E4
V_PAD = W_ROWS * 4


def _pre_body(wt_ref, out_ref):
    x = wt_ref[...]
    zs = []
    for h in range(4):
        xh = x[:, h * GRP:(h + 1) * GRP]
        xx = jnp.concatenate(
            [xh[:, q * E4:(q + 1) * E4] for q in range(4)], axis=0
        )
        zs.append(jnp.transpose(xx))
    out_ref[...] = jnp.concatenate(zs, axis=0)


_pre_transpose = pl.pallas_call(
    _pre_body,
    grid=(PRE_GRID // 4,),
    in_specs=[pl.BlockSpec((DIM, 4 * GRP), lambda i: (0, i))],
    out_specs=pl.BlockSpec((4 * E4, 128), lambda i: (i, 0)),
    out_shape=jax.ShapeDtypeStruct((W_ROWS, 128), jnp.float32),
)

_mesh = plsc.VectorSubcoreMesh(
    core_axis_name="c", subcore_axis_name="s", num_cores=NC, num_subcores=NS
)


@functools.partial(
    pl.kernel,
    out_type=jax.ShapeDtypeStruct((N_T, 8, B_BLK, 4, DIM), jnp.float32),
    mesh=_mesh,
    compiler_params=pltpu.CompilerParams(use_tc_tiling_on_sc=False),
    scratch_types=[
        pltpu.VMEM((N_T, B_BLK), jnp.int32),
        pltpu.VMEM((T_CHUNK, B_BLK, DIM), jnp.float32),
        pltpu.VMEM((T_CHUNK, B_BLK, DIM), jnp.float32),
        pltpu.SemaphoreType.DMA,
        pltpu.SemaphoreType.DMA,
        pltpu.SemaphoreType.DMA,
        pltpu.SemaphoreType.DMA,
    ],
)
def _emb_lookup(idx_hbm, table_hbm, out_hbm, idx_v, buf0, buf1,
                g0, g1, w0, w1):
    wid = lax.axis_index("s") * NC + lax.axis_index("c")
    b0 = wid * B_BLK
    jq = wid // 4
    uq = lax.rem(wid, 4)
    pltpu.sync_copy(idx_hbm.at[:, pl.ds(b0, B_BLK)], idx_v)

    @pl.loop(0, N_T)
    def _fix(r):
        for c in range(B_BLK // 16):
            e = idx_v[r, pl.ds(c * 16, 16)]
            m = (e & ~(GRP - 1)) | ((e & (E4 - 1)) << 2) | ((e >> 11) & 3)
            idx_v[r, pl.ds(c * 16, 16)] = m

    def fire(c, buf, sem):
        for tt in range(T_CHUNK):
            pltpu.async_copy(
                table_hbm.at[idx_v.at[c * T_CHUNK + tt]], buf.at[tt], sem
            )

    def drain(c, buf, sem):
        for tt in range(T_CHUNK):
            pltpu.make_async_copy(
                table_hbm.at[idx_v.at[c * T_CHUNK + tt]], buf.at[tt], sem
            ).wait()

    def wb_copy(c, buf, sem):
        return pltpu.make_async_copy(
            buf,
            out_hbm.at[pl.ds(c * T_CHUNK, T_CHUNK), jq, :, uq, :],
            sem,
        )

    def writeback(c, buf, sem):
        wb_copy(c, buf, sem).start()

    def drain_wb(c, buf, sem):
        wb_copy(c, buf, sem).wait()

    fire(0, buf0, g0)
    fire(1, buf1, g1)

    @pl.loop(0, NSTEPS // 2)
    def _pair(j):
        c0 = 2 * j
        drain(c0, buf0, g0)
        writeback(c0, buf0, w0)
        drain_wb(c0, buf0, w0)

        @pl.when(c0 + 2 < NSTEPS)
        def _():
            fire(c0 + 2, buf0, g0)

        drain(c0 + 1, buf1, g1)
        writeback(c0 + 1, buf1, w1)
        drain_wb(c0 + 1, buf1, w1)

        @pl.when(c0 + 3 < NSTEPS)
        def _():
            fire(c0 + 3, buf1, g1)


def _post_body(x_ref, out_ref):
    z = jnp.transpose(x_ref[...])
    for tt in range(4):
        y = jnp.concatenate(
            [z[32 * u:32 * (u + 1), 1024 * tt + 128 * j:1024 * tt + 128 * (j + 1)]
             for j in range(8) for u in range(4)],
            axis=1,
        )
        out_ref[tt] = y


_post_transpose = pl.pallas_call(
    _post_body,
    grid=(N_T // 4,),
    in_specs=[pl.BlockSpec((4096, 128), lambda t: (t, 0))],
    out_specs=pl.BlockSpec((4, DIM, N_B), lambda t: (t, 0, 0)),
    out_shape=jax.ShapeDtypeStruct((N_T, DIM, N_B), jnp.float32),
)


def kernel(tokens, weight):
    tok_t = tokens.T.astype(jnp.int32)
    w128 = _pre_transpose(weight.T)
    g4 = _emb_lookup(tok_t, w128.reshape(V_PAD, DIM))
    out_perm = _post_transpose(g4.reshape(N_T * 8 * B_BLK, 128))
    return jnp.transpose(out_perm, (2, 0, 1))


# speedup vs baseline: 3.9953x; 1.0619x over previous
import functools

import jax
import jax.numpy as jnp
from jax import lax
from jax.experimental import pallas as pl
from jax.experimental.pallas import tpu as pltpu
from jax.experimental.pallas import tpu_sc as plsc

NUM_EMB = 1000000
DIM = 32
N_B = 4096
N_T = 200
NC, NS = 2, 16
NW = NC * NS
B_BLK = N_B // NW
T_CHUNK = 10
NSTEPS = N_T // T_CHUNK

E4 = 2048
GRP = 4 * E4
PRE_GRID = 8 * (-(-NUM_EMB // (8 * GRP)))
W_ROWS = PRE_GRID * E4
V_PAD = W_ROWS * 4


def _pre_body(wt_ref, out_ref):
    x = wt_ref[...]
    zs = []
    for h in range(8):
        xh = x[:, h * GRP:(h + 1) * GRP]
        xx = jnp.concatenate(
            [xh[:, q * E4:(q + 1) * E4] for q in range(4)], axis=0
        )
        zs.append(jnp.transpose(xx))
    out_ref[...] = jnp.concatenate(zs, axis=0)


_pre_transpose = pl.pallas_call(
    _pre_body,
    grid=(PRE_GRID // 8,),
    in_specs=[pl.BlockSpec((DIM, 8 * GRP), lambda i: (0, i))],
    out_specs=pl.BlockSpec((8 * E4, 128), lambda i: (i, 0)),
    out_shape=jax.ShapeDtypeStruct((W_ROWS, 128), jnp.float32),
)

_mesh = plsc.VectorSubcoreMesh(
    core_axis_name="c", subcore_axis_name="s", num_cores=NC, num_subcores=NS
)


@functools.partial(
    pl.kernel,
    out_type=jax.ShapeDtypeStruct((N_T, 8, B_BLK, 4, DIM), jnp.float32),
    mesh=_mesh,
    compiler_params=pltpu.CompilerParams(use_tc_tiling_on_sc=False),
    scratch_types=[
        pltpu.VMEM((N_T, B_BLK), jnp.int32),
        pltpu.VMEM((T_CHUNK, B_BLK, DIM), jnp.float32),
        pltpu.VMEM((T_CHUNK, B_BLK, DIM), jnp.float32),
        pltpu.SemaphoreType.DMA,
        pltpu.SemaphoreType.DMA,
        pltpu.SemaphoreType.DMA,
        pltpu.SemaphoreType.DMA,
    ],
)
def _emb_lookup(idx_hbm, table_hbm, out_hbm, idx_v, buf0, buf1,
                g0, g1, w0, w1):
    wid = lax.axis_index("s") * NC + lax.axis_index("c")
    b0 = wid * B_BLK
    jq = wid // 4
    uq = lax.rem(wid, 4)
    pltpu.sync_copy(idx_hbm.at[:, pl.ds(b0, B_BLK)], idx_v)

    @pl.loop(0, N_T)
    def _fix(r):
        for c in range(B_BLK // 16):
            e = idx_v[r, pl.ds(c * 16, 16)]
            m = (e & ~(GRP - 1)) | ((e & (E4 - 1)) << 2) | ((e >> 11) & 3)
            idx_v[r, pl.ds(c * 16, 16)] = m

    def fire(c, buf, sem):
        for tt in range(T_CHUNK):
            pltpu.async_copy(
                table_hbm.at[idx_v.at[c * T_CHUNK + tt]], buf.at[tt], sem
            )

    def drain(c, buf, sem):
        for tt in range(T_CHUNK):
            pltpu.make_async_copy(
                table_hbm.at[idx_v.at[c * T_CHUNK + tt]], buf.at[tt], sem
            ).wait()

    def wb_copy(c, buf, sem):
        return pltpu.make_async_copy(
            buf,
            out_hbm.at[pl.ds(c * T_CHUNK, T_CHUNK), jq, :, uq, :],
            sem,
        )

    def writeback(c, buf, sem):
        wb_copy(c, buf, sem).start()

    def drain_wb(c, buf, sem):
        wb_copy(c, buf, sem).wait()

    fire(0, buf0, g0)
    fire(1, buf1, g1)

    @pl.loop(0, NSTEPS // 2)
    def _pair(j):
        c0 = 2 * j
        drain(c0, buf0, g0)
        writeback(c0, buf0, w0)
        drain_wb(c0, buf0, w0)

        @pl.when(c0 + 2 < NSTEPS)
        def _():
            fire(c0 + 2, buf0, g0)

        drain(c0 + 1, buf1, g1)
        writeback(c0 + 1, buf1, w1)
        drain_wb(c0 + 1, buf1, w1)

        @pl.when(c0 + 3 < NSTEPS)
        def _():
            fire(c0 + 3, buf1, g1)


def _post_body(x_ref, out_ref):
    z = jnp.transpose(x_ref[...])
    for tt in range(8):
        y = jnp.concatenate(
            [z[32 * u:32 * (u + 1), 1024 * tt + 128 * j:1024 * tt + 128 * (j + 1)]
             for j in range(8) for u in range(4)],
            axis=1,
        )
        out_ref[tt] = y


_post_transpose = pl.pallas_call(
    _post_body,
    grid=(N_T // 8,),
    in_specs=[pl.BlockSpec((8192, 128), lambda t: (t, 0))],
    out_specs=pl.BlockSpec((8, DIM, N_B), lambda t: (t, 0, 0)),
    out_shape=jax.ShapeDtypeStruct((N_T, DIM, N_B), jnp.float32),
)


def kernel(tokens, weight):
    tok_t = tokens.T.astype(jnp.int32)
    w128 = _pre_transpose(weight.T)
    g4 = _emb_lookup(tok_t, w128.reshape(V_PAD, DIM))
    out_perm = _post_transpose(g4.reshape(N_T * 8 * B_BLK, 128))
    return jnp.transpose(out_perm, (2, 0, 1))
